# Initial kernel scaffold; baseline (speedup 1.0000x reference)
import jax
import jax.numpy as jnp
from jax.experimental import pallas as pl


def kernel(confidence, predicted_locations, labels, gt_locations):
    raise NotImplementedError("write your pallas kernel here")


# speedup vs baseline: 1.5264x; 1.5264x over previous
import functools

import jax
import jax.numpy as jnp
from jax.experimental import pallas as pl

_NEG_POS_RATIO = 3
_ROWS = 8
_PBLK = 1152


def _row_kernel(nprior, pblk, conf_ref, lab_ref, pred_ref, gt_ref,
                mining_ref, ce_ref, sl1_ref):
    x = conf_ref[...]
    lab = lab_ref[...]
    m = jnp.max(x, axis=2, keepdims=True)
    e = jnp.exp(x - m)
    s = jnp.sum(e, axis=2, keepdims=True)
    lse = m + jnp.log(s)
    mining_ref[...] = (lse - x[:, :, 0:1])[:, :, 0]
    c_iota = jax.lax.broadcasted_iota(jnp.int32, x.shape, 2)
    picked = jnp.sum(jnp.where(c_iota == lab[:, :, None], x, 0.0),
                     axis=2, keepdims=True)
    ce_ref[...] = (lse - picked)[:, :, 0]
    d = pred_ref[...] - gt_ref[...]
    ad = jnp.abs(d)
    sl1 = jnp.where(ad < 1.0, 0.5 * d * d, ad - 0.5)
    s4 = jnp.sum(sl1, axis=2)
    p_iota = jax.lax.broadcasted_iota(jnp.int32, lab.shape, 1)
    valid = (pl.program_id(1) * pblk + p_iota) < nprior
    part = jnp.sum(jnp.where(valid & (lab > 0), s4, 0.0))

    @pl.when(pl.program_id(1) == 0)
    def _init():
        sl1_ref[...] = jnp.zeros((1, 1, 128), jnp.float32)

    sl1_ref[...] += jnp.full((1, 1, 128), part, jnp.float32)


def _select_kernel(mining_ref, ce_ref, lab_ref, sl1_ref, loc_out, cls_out):
    lab = lab_ref[...]
    pos = lab > 0
    k = _NEG_POS_RATIO * jnp.sum(pos.astype(jnp.int32), axis=1, keepdims=True)
    mining = mining_ref[...]
    key = jax.lax.bitcast_convert_type(mining, jnp.int32)
    key = jnp.where(pos, -1, key)
    t = jnp.zeros_like(k)
    for bit in range(30, -1, -1):
        cand = t | (1 << bit)
        cnt = jnp.sum((key >= cand).astype(jnp.int32), axis=1, keepdims=True)
        t = jnp.where(cnt >= k, cand, t)
    gt_mask = key > t
    need_eq = k - jnp.sum(gt_mask.astype(jnp.int32), axis=1, keepdims=True)
    eq = key == t
    idx = jax.lax.broadcasted_iota(jnp.int32, lab.shape, 1)
    it = jnp.zeros_like(k)
    for bit in range(14, -1, -1):
        cand = it | (1 << bit)
        c = jnp.sum((eq & (idx < cand)).astype(jnp.int32),
                    axis=1, keepdims=True)
        it = jnp.where(c <= need_eq, cand, it)
    sel = pos | gt_mask | (eq & (idx < it))
    cls_sum = jnp.sum(ce_ref[...] * sel.astype(jnp.float32))
    np_tot = jnp.sum(pos.astype(jnp.float32))
    loc_out[...] = (jnp.sum(sl1_ref[...][:, :, 0:1]) / np_tot).reshape(1, 1)
    cls_out[...] = (cls_sum / np_tot).reshape(1, 1)


def kernel(confidence, predicted_locations, labels, gt_locations):
    B, P, C = confidence.shape
    R = _ROWS
    Pb = _PBLK
    nb = B // R
    np_ = pl.cdiv(P, Pb)
    mining, ce, sl1 = pl.pallas_call(
        functools.partial(_row_kernel, P, Pb),
        grid=(nb, np_),
        in_specs=[
            pl.BlockSpec((R, Pb, C), lambda i, j: (i, j, 0)),
            pl.BlockSpec((R, Pb), lambda i, j: (i, j)),
            pl.BlockSpec((R, Pb, 4), lambda i, j: (i, j, 0)),
            pl.BlockSpec((R, Pb, 4), lambda i, j: (i, j, 0)),
        ],
        out_specs=[
            pl.BlockSpec((R, Pb), lambda i, j: (i, j)),
            pl.BlockSpec((R, Pb), lambda i, j: (i, j)),
            pl.BlockSpec((1, 1, 128), lambda i, j: (i, 0, 0)),
        ],
        out_shape=[
            jax.ShapeDtypeStruct((B, P), jnp.float32),
            jax.ShapeDtypeStruct((B, P), jnp.float32),
            jax.ShapeDtypeStruct((nb, 1, 128), jnp.float32),
        ],
    )(confidence, labels, predicted_locations, gt_locations)

    loc_loss, cls_loss = pl.pallas_call(
        _select_kernel,
        out_shape=[
            jax.ShapeDtypeStruct((1, 1), jnp.float32),
            jax.ShapeDtypeStruct((1, 1), jnp.float32),
        ],
    )(mining, ce, labels, sl1)

    return (loc_loss[0, 0], cls_loss[0, 0])


# speedup vs baseline: 22.5217x; 14.7546x over previous
import functools

import jax
import jax.numpy as jnp
from jax.experimental import pallas as pl

_NEG_POS_RATIO = 3
_PBLK = 1152


def _row_kernel(nprior, pblk, conf_ref, lab_ref, lab3_ref, pred_ref, gt_ref,
                mining_ref, ce_ref, sl1_ref):
    x = conf_ref[...]
    lab = lab_ref[...]
    m = jnp.max(x, axis=0)
    s = jnp.sum(jnp.exp(x - m[None]), axis=0)
    lse = m + jnp.log(s)
    mining_ref[...] = lse - x[0]
    c_iota = jax.lax.broadcasted_iota(jnp.int32, x.shape, 0)
    picked = jnp.sum(jnp.where(c_iota == lab[None], x, 0.0), axis=0)
    ce_ref[...] = lse - picked
    d = pred_ref[...] - gt_ref[...]
    ad = jnp.abs(d)
    sl1 = jnp.where(ad < 1.0, 0.5 * d * d, ad - 0.5)
    s4 = jnp.sum(sl1, axis=1, keepdims=True)
    lab3 = lab3_ref[...]
    p_iota = jax.lax.broadcasted_iota(jnp.int32, lab3.shape, 2)
    keep = (lab3 > 0) & ((pl.program_id(0) * pblk + p_iota) < nprior)
    part = jnp.sum(jnp.where(keep, s4, 0.0))

    @pl.when(pl.program_id(0) == 0)
    def _init():
        sl1_ref[...] = jnp.zeros((1, 1, 128), jnp.float32)

    sl1_ref[...] += jnp.full((1, 1, 128), part, jnp.float32)


def _select_kernel(mining_ref, ce_ref, lab_ref, sl1_ref, loc_out, cls_out):
    lab = lab_ref[...]
    pos = lab > 0
    k = _NEG_POS_RATIO * jnp.sum(pos.astype(jnp.int32), axis=1, keepdims=True)
    mining = mining_ref[...]
    key = jax.lax.bitcast_convert_type(mining, jnp.int32)
    key = jnp.where(pos, -1, key)
    t = jnp.zeros_like(k)
    for bit in range(30, -1, -1):
        cand = t | (1 << bit)
        cnt = jnp.sum((key >= cand).astype(jnp.int32), axis=1, keepdims=True)
        t = jnp.where(cnt >= k, cand, t)
    gt_mask = key > t
    need_eq = k - jnp.sum(gt_mask.astype(jnp.int32), axis=1, keepdims=True)
    eq = key == t
    idx = jax.lax.broadcasted_iota(jnp.int32, lab.shape, 1)
    it = jnp.zeros_like(k)
    for bit in range(14, -1, -1):
        cand = it | (1 << bit)
        c = jnp.sum((eq & (idx < cand)).astype(jnp.int32),
                    axis=1, keepdims=True)
        it = jnp.where(c <= need_eq, cand, it)
    sel = pos | gt_mask | (eq & (idx < it))
    cls_sum = jnp.sum(ce_ref[...] * sel.astype(jnp.float32))
    np_tot = jnp.sum(pos.astype(jnp.float32))
    loc_out[...] = (jnp.sum(sl1_ref[...][:, :, 0:1]) / np_tot).reshape(1, 1)
    cls_out[...] = (cls_sum / np_tot).reshape(1, 1)


def kernel(confidence, predicted_locations, labels, gt_locations):
    B, P, C = confidence.shape
    Pb = _PBLK
    np_ = pl.cdiv(P, Pb)
    conf_t = jnp.transpose(confidence, (2, 0, 1))
    pred_t = jnp.transpose(predicted_locations, (0, 2, 1))
    gt_t = jnp.transpose(gt_locations, (0, 2, 1))
    labels3 = labels.reshape(B, 1, P)
    mining, ce, sl1 = pl.pallas_call(
        functools.partial(_row_kernel, P, Pb),
        grid=(np_,),
        in_specs=[
            pl.BlockSpec((C, B, Pb), lambda j: (0, 0, j)),
            pl.BlockSpec((B, Pb), lambda j: (0, j)),
            pl.BlockSpec((B, 1, Pb), lambda j: (0, 0, j)),
            pl.BlockSpec((B, 4, Pb), lambda j: (0, 0, j)),
            pl.BlockSpec((B, 4, Pb), lambda j: (0, 0, j)),
        ],
        out_specs=[
            pl.BlockSpec((B, Pb), lambda j: (0, j)),
            pl.BlockSpec((B, Pb), lambda j: (0, j)),
            pl.BlockSpec((1, 1, 128), lambda j: (0, 0, 0)),
        ],
        out_shape=[
            jax.ShapeDtypeStruct((B, P), jnp.float32),
            jax.ShapeDtypeStruct((B, P), jnp.float32),
            jax.ShapeDtypeStruct((1, 1, 128), jnp.float32),
        ],
    )(conf_t, labels, labels3, pred_t, gt_t)

    loc_loss, cls_loss = pl.pallas_call(
        _select_kernel,
        out_shape=[
            jax.ShapeDtypeStruct((1, 1), jnp.float32),
            jax.ShapeDtypeStruct((1, 1), jnp.float32),
        ],
    )(mining, ce, labels, sl1)

    return (loc_loss[0, 0], cls_loss[0, 0])


# speedup vs baseline: 23.6681x; 1.0509x over previous
import functools

import jax
import jax.numpy as jnp
from jax.experimental import pallas as pl
from jax.experimental.pallas import tpu as pltpu

_NEG_POS_RATIO = 3
_PBLK = 1152


def _kernel(nprior, pblk, nchunk,
            conf_ref, lab_ref, lab3_ref, pred_ref, gt_ref,
            loc_out, cls_out,
            mining_s, ce_s, lab_s, sl1_s):
    j = pl.program_id(0)

    @pl.when(j == 0)
    def _init():
        sl1_s[...] = jnp.zeros_like(sl1_s)

    @pl.when(j < nchunk)
    def _chunk():
        x = conf_ref[...]
        lab = lab_ref[...]
        m = jnp.max(x, axis=0)
        s = jnp.sum(jnp.exp(x - m[None]), axis=0)
        lse = m + jnp.log(s)
        mining_s[j] = lse - x[0]
        c_iota = jax.lax.broadcasted_iota(jnp.int32, x.shape, 0)
        picked = jnp.sum(jnp.where(c_iota == lab[None], x, 0.0), axis=0)
        ce_s[j] = lse - picked
        lab_s[j] = lab
        d = pred_ref[...] - gt_ref[...]
        ad = jnp.abs(d)
        sl1 = jnp.where(ad < 1.0, 0.5 * d * d, ad - 0.5)
        s4 = jnp.sum(sl1, axis=1, keepdims=True)
        lab3 = lab3_ref[...]
        p_iota = jax.lax.broadcasted_iota(jnp.int32, lab3.shape, 2)
        keep = (lab3 > 0) & ((j * pblk + p_iota) < nprior)
        sl1_s[...] += jnp.full(sl1_s.shape, jnp.sum(jnp.where(keep, s4, 0.0)))

    @pl.when(j == nchunk)
    def _select():
        lab = lab_s[...]
        shape3 = lab.shape
        c_iota = jax.lax.broadcasted_iota(jnp.int32, shape3, 0)
        l_iota = jax.lax.broadcasted_iota(jnp.int32, shape3, 2)
        gidx = c_iota * pblk + l_iota
        valid = gidx < nprior
        pos = (lab > 0) & valid
        k = _NEG_POS_RATIO * jnp.sum(pos.astype(jnp.int32),
                                     axis=(0, 2), keepdims=True)
        key = jax.lax.bitcast_convert_type(mining_s[...], jnp.int32)
        key = jnp.where(pos | ~valid, -1, key)
        t = jnp.zeros_like(k)
        for bit in range(30, -1, -1):
            cand = t | (1 << bit)
            cnt = jnp.sum((key >= cand).astype(jnp.int32),
                          axis=(0, 2), keepdims=True)
            t = jnp.where(cnt >= k, cand, t)
        gt_mask = key > t
        need_eq = k - jnp.sum(gt_mask.astype(jnp.int32),
                              axis=(0, 2), keepdims=True)
        eq = key == t
        it = jnp.zeros_like(k)
        for bit in range(14, -1, -1):
            cand = it | (1 << bit)
            c = jnp.sum((eq & (gidx < cand)).astype(jnp.int32),
                        axis=(0, 2), keepdims=True)
            it = jnp.where(c <= need_eq, cand, it)
        sel = pos | gt_mask | (eq & (gidx < it))
        cls_sum = jnp.sum(jnp.where(sel, ce_s[...], 0.0))
        np_tot = jnp.sum(pos.astype(jnp.float32))
        loc_out[...] = (jnp.sum(sl1_s[...][:, 0:1]) / np_tot).reshape(1, 1)
        cls_out[...] = (cls_sum / np_tot).reshape(1, 1)


def kernel(confidence, predicted_locations, labels, gt_locations):
    B, P, C = confidence.shape
    Pb = _PBLK
    np_ = pl.cdiv(P, Pb)
    conf_t = jnp.transpose(confidence, (2, 0, 1))
    pred_t = jnp.transpose(predicted_locations, (0, 2, 1))
    gt_t = jnp.transpose(gt_locations, (0, 2, 1))
    labels3 = labels.reshape(B, 1, P)
    last = np_ - 1
    loc_loss, cls_loss = pl.pallas_call(
        functools.partial(_kernel, P, Pb, np_),
        grid=(np_ + 1,),
        in_specs=[
            pl.BlockSpec((C, B, Pb), lambda j: (0, 0, jnp.minimum(j, last))),
            pl.BlockSpec((B, Pb), lambda j: (0, jnp.minimum(j, last))),
            pl.BlockSpec((B, 1, Pb), lambda j: (0, 0, jnp.minimum(j, last))),
            pl.BlockSpec((B, 4, Pb), lambda j: (0, 0, jnp.minimum(j, last))),
            pl.BlockSpec((B, 4, Pb), lambda j: (0, 0, jnp.minimum(j, last))),
        ],
        out_specs=[
            pl.BlockSpec((1, 1), lambda j: (0, 0)),
            pl.BlockSpec((1, 1), lambda j: (0, 0)),
        ],
        out_shape=[
            jax.ShapeDtypeStruct((1, 1), jnp.float32),
            jax.ShapeDtypeStruct((1, 1), jnp.float32),
        ],
        scratch_shapes=[
            pltpu.VMEM((np_, B, Pb), jnp.float32),
            pltpu.VMEM((np_, B, Pb), jnp.float32),
            pltpu.VMEM((np_, B, Pb), jnp.int32),
            pltpu.VMEM((1, 128), jnp.float32),
        ],
    )(conf_t, labels, labels3, pred_t, gt_t)

    return (loc_loss[0, 0], cls_loss[0, 0])
